# baseline (speedup 1.0000x reference)
import functools

import jax
import jax.numpy as jnp
from jax import lax
from jax.experimental import pallas as pl
from jax.experimental.pallas import tpu as pltpu
from jax.experimental.pallas import tpu_sc as plsc

N = 10000
E = 320000
D = 128

NC = 2
NS = 16
NW = NC * NS
CHUNK = 128
NCH = 80
EPW = NCH * CHUNK
EPAD = NW * EPW
NPAD = 10240
RPS = NPAD // NS

_mesh = plsc.VectorSubcoreMesh(core_axis_name="c", subcore_axis_name="s")


@functools.partial(
    pl.kernel,
    out_type=jax.ShapeDtypeStruct((NC, NPAD, D), jnp.float32),
    mesh=_mesh,
    scratch_types=[
        pltpu.VMEM((NCH, CHUNK), jnp.int32),
        pltpu.VMEM((CHUNK, D), jnp.float32),
        pltpu.VMEM_SHARED((NPAD, D), jnp.float32),
        pltpu.SemaphoreType.DMA,
        pltpu.SemaphoreType.DMA,
        pltpu.SemaphoreType.DMA,
    ],
)
def _sc_degree(dst_hbm, out_hbm, dst_v, ones_v, acc_sh, sem, sem_a, sem_b):
    cid = lax.axis_index("c")
    sid = lax.axis_index("s")
    wid = cid * NS + sid

    pltpu.async_copy(dst_hbm.at[wid], dst_v, sem)

    @pl.loop(0, CHUNK)
    def _(i):
        @pl.loop(0, D, step=16)
        def _(k):
            ones_v[i, pl.ds(k, 16)] = jnp.zeros((16,), jnp.float32)

    @pl.loop(0, RPS, step=CHUNK)
    def _(r):
        pltpu.sync_copy(ones_v, acc_sh.at[pl.ds(sid * RPS + r, CHUNK)])

    @pl.loop(0, CHUNK)
    def _(i):
        @pl.loop(0, D, step=16)
        def _(k):
            ones_v[i, pl.ds(k, 16)] = jnp.full((16,), 1.0, jnp.float32)

    pltpu.make_async_copy(dst_hbm.at[wid], dst_v, sem).wait()
    plsc.subcore_barrier()

    pltpu.async_copy(ones_v, acc_sh.at[dst_v.at[0]], sem_a, add=True)

    @pl.loop(0, NCH, step=2)
    def _(j):
        pltpu.async_copy(ones_v, acc_sh.at[dst_v.at[j + 1]], sem_b, add=True)
        pltpu.make_async_copy(ones_v, acc_sh.at[dst_v.at[j]], sem_a).wait()

        @pl.when(j + 2 < NCH)
        def _():
            pltpu.async_copy(ones_v, acc_sh.at[dst_v.at[j + 2]], sem_a, add=True)

        pltpu.make_async_copy(ones_v, acc_sh.at[dst_v.at[j + 1]], sem_b).wait()

    plsc.subcore_barrier()
    pltpu.sync_copy(acc_sh.at[pl.ds(sid * RPS, RPS)],
                    out_hbm.at[cid, pl.ds(sid * RPS, RPS)])


@functools.partial(
    pl.kernel,
    out_type=jax.ShapeDtypeStruct((NC, NPAD, D), jnp.float32),
    mesh=_mesh,
    scratch_types=[
        pltpu.VMEM((NCH // 2, CHUNK), jnp.int32),
        pltpu.VMEM((NCH // 2, CHUNK), jnp.int32),
        pltpu.VMEM((CHUNK, D), jnp.float32),
        pltpu.VMEM((CHUNK, D), jnp.float32),
        pltpu.VMEM_SHARED((NPAD, D), jnp.float32),
        pltpu.SemaphoreType.DMA,
        pltpu.SemaphoreType.DMA,
        pltpu.SemaphoreType.DMA,
        pltpu.SemaphoreType.DMA,
        pltpu.SemaphoreType.DMA,
    ],
)
def _sc_spmm(g_hbm, src_hbm, dst_hbm, out_hbm, src_v, dst_v, rows_a, rows_b,
             acc_sh, sem_a, sem_b, sem_sa, sem_sb, sem_i):
    cid = lax.axis_index("c")
    sid = lax.axis_index("s")
    wid = cid * NS + sid
    nchh = NCH // 2

    @pl.loop(0, CHUNK)
    def _(i):
        @pl.loop(0, D, step=16)
        def _(k):
            rows_a[i, pl.ds(k, 16)] = jnp.zeros((16,), jnp.float32)

    @pl.loop(0, RPS, step=CHUNK)
    def _(r):
        pltpu.sync_copy(rows_a, acc_sh.at[pl.ds(sid * RPS + r, CHUNK)])

    plsc.subcore_barrier()

    for half in range(2):
        pltpu.async_copy(src_hbm.at[wid, pl.ds(half * nchh, nchh)],
                         src_v, sem_i).wait()
        pltpu.async_copy(dst_hbm.at[wid, pl.ds(half * nchh, nchh)],
                         dst_v, sem_i).wait()
        pltpu.async_copy(g_hbm.at[src_v.at[0]], rows_a, sem_a)
        pltpu.async_copy(g_hbm.at[src_v.at[1]], rows_b, sem_b)

        @pl.loop(0, nchh, step=2)
        def _(j):
            pltpu.make_async_copy(g_hbm.at[src_v.at[j]], rows_a, sem_a).wait()
            pltpu.async_copy(rows_a, acc_sh.at[dst_v.at[j]], sem_sa, add=True)
            pltpu.make_async_copy(g_hbm.at[src_v.at[j + 1]], rows_b, sem_b).wait()
            pltpu.async_copy(rows_b, acc_sh.at[dst_v.at[j + 1]], sem_sb, add=True)
            pltpu.make_async_copy(rows_a, acc_sh.at[dst_v.at[j]], sem_sa).wait()

            @pl.when(j + 2 < nchh)
            def _():
                pltpu.async_copy(g_hbm.at[src_v.at[j + 2]], rows_a, sem_a)

            pltpu.make_async_copy(rows_b, acc_sh.at[dst_v.at[j + 1]], sem_sb).wait()

            @pl.when(j + 3 < nchh)
            def _():
                pltpu.async_copy(g_hbm.at[src_v.at[j + 3]], rows_b, sem_b)

    plsc.subcore_barrier()
    pltpu.sync_copy(acc_sh.at[pl.ds(sid * RPS, RPS)],
                    out_hbm.at[cid, pl.ds(sid * RPS, RPS)])


BM = 1000


def _tc_matmul(x, W):
    def body(x_ref, w_ref, o_ref):
        o_ref[...] = jnp.dot(x_ref[...], w_ref[...],
                             preferred_element_type=jnp.float32)

    return pl.pallas_call(
        body,
        grid=(N // BM,),
        in_specs=[pl.BlockSpec((BM, D), lambda i: (i, 0)),
                  pl.BlockSpec((D, D), lambda i: (0, 0))],
        out_specs=pl.BlockSpec((BM, D), lambda i: (i, 0)),
        out_shape=jax.ShapeDtypeStruct((N, D), jnp.float32),
    )(x, W)


def _tc_scale(h1, d0, d1):
    def body(h_ref, d0_ref, d1_ref, g_ref, dinv_ref):
        dinv = lax.rsqrt(d0_ref[...] + d1_ref[...] + 1.0)
        dinv_ref[...] = dinv
        g_ref[...] = h_ref[...] * dinv

    return pl.pallas_call(
        body,
        grid=(N // BM,),
        in_specs=[pl.BlockSpec((BM, D), lambda i: (i, 0)),
                  pl.BlockSpec((BM, 1), lambda i: (i, 0)),
                  pl.BlockSpec((BM, 1), lambda i: (i, 0))],
        out_specs=[pl.BlockSpec((BM, D), lambda i: (i, 0)),
                   pl.BlockSpec((BM, 1), lambda i: (i, 0))],
        out_shape=[jax.ShapeDtypeStruct((N, D), jnp.float32),
                   jax.ShapeDtypeStruct((N, 1), jnp.float32)],
    )(h1, d0, d1)


def _tc_layer(p0, p1, g, dinv, b, W):
    def body(p0_ref, p1_ref, g_ref, dinv_ref, b_ref, w_ref, o_ref):
        y = (p0_ref[...] + p1_ref[...] + g_ref[...]) * dinv_ref[...] + b_ref[...]
        y = jnp.maximum(y, 0.0)
        o_ref[...] = jnp.dot(y, w_ref[...],
                             preferred_element_type=jnp.float32) * dinv_ref[...]

    return pl.pallas_call(
        body,
        grid=(N // BM,),
        in_specs=[pl.BlockSpec((BM, D), lambda i: (i, 0)),
                  pl.BlockSpec((BM, D), lambda i: (i, 0)),
                  pl.BlockSpec((BM, D), lambda i: (i, 0)),
                  pl.BlockSpec((BM, 1), lambda i: (i, 0)),
                  pl.BlockSpec((1, D), lambda i: (0, 0)),
                  pl.BlockSpec((D, D), lambda i: (0, 0))],
        out_specs=pl.BlockSpec((BM, D), lambda i: (i, 0)),
        out_shape=jax.ShapeDtypeStruct((N, D), jnp.float32),
    )(p0, p1, g, dinv, b, W)


def _tc_final(q0, q1, g, dinv, b2, Wl, bl):
    nb = N // BM

    def body(q0_ref, q1_ref, g_ref, dinv_ref, b2_ref, wl_ref, bl_ref, o_ref,
             acc_ref):
        i = pl.program_id(0)
        y = (q0_ref[...] + q1_ref[...] + g_ref[...]) * dinv_ref[...] + b2_ref[...]
        y = jnp.maximum(y, 0.0)
        s = jnp.sum(y, axis=0, keepdims=True)

        @pl.when(i == 0)
        def _():
            acc_ref[...] = s

        @pl.when(i > 0)
        def _():
            acc_ref[...] += s

        @pl.when(i == nb - 1)
        def _():
            m = acc_ref[...] * (1.0 / N)
            o_ref[...] = jnp.dot(m, wl_ref[...],
                                 preferred_element_type=jnp.float32) + bl_ref[...]

    return pl.pallas_call(
        body,
        grid=(nb,),
        in_specs=[pl.BlockSpec((BM, D), lambda i: (i, 0)),
                  pl.BlockSpec((BM, D), lambda i: (i, 0)),
                  pl.BlockSpec((BM, D), lambda i: (i, 0)),
                  pl.BlockSpec((BM, 1), lambda i: (i, 0)),
                  pl.BlockSpec((1, D), lambda i: (0, 0)),
                  pl.BlockSpec((D, D), lambda i: (0, 0)),
                  pl.BlockSpec((1, D), lambda i: (0, 0))],
        out_specs=pl.BlockSpec((1, D), lambda i: (0, 0)),
        out_shape=jax.ShapeDtypeStruct((1, D), jnp.float32),
        scratch_shapes=[pltpu.VMEM((1, D), jnp.float32)],
    )(q0, q1, g, dinv, b2, Wl, bl)


def kernel(x, edge_index, W1, b1, W2, b2, Wl, bl):
    src = edge_index[0]
    dst = edge_index[1]

    pad = EPAD - E
    pad_iota = lax.iota(jnp.int32, pad)
    src_p = jnp.concatenate([src, pad_iota % 128]).reshape(NW, NCH, CHUNK)
    dst_p = jnp.concatenate([dst, N + (pad_iota % (NPAD - N))]).reshape(NW, NCH, CHUNK)

    degp = _sc_degree(dst_p)
    h1 = _tc_matmul(x, W1)
    g1, dinv = _tc_scale(h1, degp[0, :N, :1], degp[1, :N, :1])
    p = _sc_spmm(g1, src_p, dst_p)
    g2 = _tc_layer(p[0, :N], p[1, :N], g1, dinv, b1.reshape(1, D), W2)
    q = _sc_spmm(g2, src_p, dst_p)
    out = _tc_final(q[0, :N], q[1, :N], g2, dinv, b2.reshape(1, D),
                    Wl, bl.reshape(1, D))
    return out[0]


# speedup vs baseline: 1.1896x; 1.1896x over previous
import dataclasses
import functools

import jax
import jax.numpy as jnp
from jax import lax
from jax.experimental import pallas as pl
from jax.experimental.pallas import tpu as pltpu
from jax.experimental.pallas import tpu_sc as plsc

N = 10000
E = 320000
D = 128

NC = 2
NS = 16
NW = NC * NS
CHUNK = 128
NCH = 80
EPW = NCH * CHUNK
EPAD = NW * EPW
NPAD = 10240
RPS = NPAD // NS

_mesh = plsc.VectorSubcoreMesh(core_axis_name="c", subcore_axis_name="s")

_cp_no_layout = pltpu.CompilerParams()
if "needs_layout_passes" in pltpu.CompilerParams.__dataclass_fields__:
    _cp_no_layout = dataclasses.replace(_cp_no_layout, needs_layout_passes=False)


def _shift_up(x, d, iota, sentinel):
    g = jax.lax.gather(
        x, jnp.maximum(iota - d, 0).reshape(16, 1),
        jax.lax.GatherDimensionNumbers(offset_dims=(), collapsed_slice_dims=(0,),
                                       start_index_map=(0,)),
        (1,), mode=jax.lax.GatherScatterMode.PROMISE_IN_BOUNDS)
    return jnp.where(iota >= d, g, sentinel)


@functools.partial(
    pl.kernel,
    out_type=jax.ShapeDtypeStruct((NC, NPAD), jnp.float32),
    mesh=_mesh,
    scratch_types=[
        pltpu.VMEM((NCH, CHUNK), jnp.int32),
        pltpu.VMEM((NPAD,), jnp.float32),
        pltpu.VMEM((NS, RPS), jnp.float32),
        pltpu.VMEM((RPS,), jnp.float32),
        pltpu.VMEM_SHARED((NS, NPAD), jnp.float32),
        pltpu.SemaphoreType.DMA,
    ],
    compiler_params=_cp_no_layout,
)
def _sc_degree(dst_hbm, out_hbm, dst_v, hist_v, blk_v, res_v, hist_sh, sem):
    cid = lax.axis_index("c")
    sid = lax.axis_index("s")
    wid = cid * NS + sid

    pltpu.async_copy(dst_hbm.at[wid], dst_v, sem)

    @pl.loop(0, NPAD, step=16)
    def _(i):
        hist_v[pl.ds(i, 16)] = jnp.zeros((16,), jnp.float32)

    pltpu.make_async_copy(dst_hbm.at[wid], dst_v, sem).wait()

    iota = lax.iota(jnp.int32, 16)
    ones = jnp.full((16,), 1.0, jnp.float32)

    @pl.loop(0, NCH)
    def _(j):
        @pl.loop(0, CHUNK, step=16)
        def _(k):
            idx = dst_v[j, pl.ds(k, 16)]
            sk, sv = plsc.sort_key_val(idx, ones)
            for dd in (1, 2, 4, 8):
                pk = _shift_up(sk, dd, iota, -1)
                pv = _shift_up(sv, dd, iota, 0.0)
                sv = sv + jnp.where(sk == pk, pv, 0.0)
            nk = jax.lax.gather(
                sk, jnp.minimum(iota + 1, 15).reshape(16, 1),
                jax.lax.GatherDimensionNumbers(
                    offset_dims=(), collapsed_slice_dims=(0,),
                    start_index_map=(0,)),
                (1,), mode=jax.lax.GatherScatterMode.PROMISE_IN_BOUNDS)
            last = jnp.where(iota == 15, jnp.int32(-2), nk) != sk
            plsc.addupdate_scatter(hist_v, [sk], sv, mask=last)

    pltpu.sync_copy(hist_v, hist_sh.at[sid])
    plsc.subcore_barrier()
    pltpu.sync_copy(hist_sh.at[:, pl.ds(sid * RPS, RPS)], blk_v)

    @pl.loop(0, RPS, step=16)
    def _(k):
        s = jnp.zeros((16,), jnp.float32)
        for h in range(NS):
            s = s + blk_v[h, pl.ds(k, 16)]
        res_v[pl.ds(k, 16)] = s

    pltpu.sync_copy(res_v, out_hbm.at[cid, pl.ds(sid * RPS, RPS)])


@functools.partial(
    pl.kernel,
    out_type=jax.ShapeDtypeStruct((NC, NPAD, D), jnp.float32),
    mesh=_mesh,
    scratch_types=[
        pltpu.VMEM((NCH // 2, CHUNK), jnp.int32),
        pltpu.VMEM((NCH // 2, CHUNK), jnp.int32),
        pltpu.VMEM((CHUNK, D), jnp.float32),
        pltpu.VMEM((CHUNK, D), jnp.float32),
        pltpu.VMEM_SHARED((NPAD, D), jnp.float32),
        pltpu.SemaphoreType.DMA,
        pltpu.SemaphoreType.DMA,
        pltpu.SemaphoreType.DMA,
    ],
)
def _sc_spmm(g_hbm, src_hbm, dst_hbm, out_hbm, src_v, dst_v, rows_a, rows_b,
             acc_sh, sem_a, sem_b, sem_i):
    cid = lax.axis_index("c")
    sid = lax.axis_index("s")
    wid = cid * NS + sid
    nchh = NCH // 2

    @pl.loop(0, CHUNK)
    def _(i):
        @pl.loop(0, D, step=16)
        def _(k):
            rows_a[i, pl.ds(k, 16)] = jnp.zeros((16,), jnp.float32)

    @pl.loop(0, RPS, step=CHUNK)
    def _(r):
        pltpu.sync_copy(rows_a, acc_sh.at[pl.ds(sid * RPS + r, CHUNK)])

    plsc.subcore_barrier()

    for half in range(2):
        pltpu.async_copy(src_hbm.at[wid, pl.ds(half * nchh, nchh)],
                         src_v, sem_i).wait()
        pltpu.async_copy(dst_hbm.at[wid, pl.ds(half * nchh, nchh)],
                         dst_v, sem_i).wait()
        pltpu.async_copy(g_hbm.at[src_v.at[0]], rows_a, sem_a)

        @pl.loop(0, nchh, step=2)
        def _(j):
            pltpu.make_async_copy(g_hbm.at[src_v.at[j]], rows_a, sem_a).wait()
            pltpu.async_copy(g_hbm.at[src_v.at[j + 1]], rows_b, sem_b)
            pltpu.sync_copy(rows_a, acc_sh.at[dst_v.at[j]], add=True)
            pltpu.make_async_copy(g_hbm.at[src_v.at[j + 1]], rows_b, sem_b).wait()

            @pl.when(j + 2 < nchh)
            def _():
                pltpu.async_copy(g_hbm.at[src_v.at[j + 2]], rows_a, sem_a)

            pltpu.sync_copy(rows_b, acc_sh.at[dst_v.at[j + 1]], add=True)

    plsc.subcore_barrier()
    pltpu.sync_copy(acc_sh.at[pl.ds(sid * RPS, RPS)],
                    out_hbm.at[cid, pl.ds(sid * RPS, RPS)])


BM = 1000


def _tc_matmul(x, W):
    def body(x_ref, w_ref, o_ref):
        o_ref[...] = jnp.dot(x_ref[...], w_ref[...],
                             preferred_element_type=jnp.float32)

    return pl.pallas_call(
        body,
        grid=(N // BM,),
        in_specs=[pl.BlockSpec((BM, D), lambda i: (i, 0)),
                  pl.BlockSpec((D, D), lambda i: (0, 0))],
        out_specs=pl.BlockSpec((BM, D), lambda i: (i, 0)),
        out_shape=jax.ShapeDtypeStruct((N, D), jnp.float32),
    )(x, W)


def _tc_scale(h1, d0, d1):
    def body(h_ref, d0_ref, d1_ref, g_ref, dinv_ref):
        dinv = lax.rsqrt(d0_ref[...] + d1_ref[...] + 1.0)
        dinv_ref[...] = dinv
        g_ref[...] = h_ref[...] * dinv

    return pl.pallas_call(
        body,
        grid=(N // BM,),
        in_specs=[pl.BlockSpec((BM, D), lambda i: (i, 0)),
                  pl.BlockSpec((BM, 1), lambda i: (i, 0)),
                  pl.BlockSpec((BM, 1), lambda i: (i, 0))],
        out_specs=[pl.BlockSpec((BM, D), lambda i: (i, 0)),
                   pl.BlockSpec((BM, 1), lambda i: (i, 0))],
        out_shape=[jax.ShapeDtypeStruct((N, D), jnp.float32),
                   jax.ShapeDtypeStruct((N, 1), jnp.float32)],
    )(h1, d0, d1)


def _tc_layer(p0, p1, g, dinv, b, W):
    def body(p0_ref, p1_ref, g_ref, dinv_ref, b_ref, w_ref, o_ref):
        y = (p0_ref[...] + p1_ref[...] + g_ref[...]) * dinv_ref[...] + b_ref[...]
        y = jnp.maximum(y, 0.0)
        o_ref[...] = jnp.dot(y, w_ref[...],
                             preferred_element_type=jnp.float32) * dinv_ref[...]

    return pl.pallas_call(
        body,
        grid=(N // BM,),
        in_specs=[pl.BlockSpec((BM, D), lambda i: (i, 0)),
                  pl.BlockSpec((BM, D), lambda i: (i, 0)),
                  pl.BlockSpec((BM, D), lambda i: (i, 0)),
                  pl.BlockSpec((BM, 1), lambda i: (i, 0)),
                  pl.BlockSpec((1, D), lambda i: (0, 0)),
                  pl.BlockSpec((D, D), lambda i: (0, 0))],
        out_specs=pl.BlockSpec((BM, D), lambda i: (i, 0)),
        out_shape=jax.ShapeDtypeStruct((N, D), jnp.float32),
    )(p0, p1, g, dinv, b, W)


def _tc_final(q0, q1, g, dinv, b2, Wl, bl):
    nb = N // BM

    def body(q0_ref, q1_ref, g_ref, dinv_ref, b2_ref, wl_ref, bl_ref, o_ref,
             acc_ref):
        i = pl.program_id(0)
        y = (q0_ref[...] + q1_ref[...] + g_ref[...]) * dinv_ref[...] + b2_ref[...]
        y = jnp.maximum(y, 0.0)
        s = jnp.sum(y, axis=0, keepdims=True)

        @pl.when(i == 0)
        def _():
            acc_ref[...] = s

        @pl.when(i > 0)
        def _():
            acc_ref[...] += s

        @pl.when(i == nb - 1)
        def _():
            m = acc_ref[...] * (1.0 / N)
            o_ref[...] = jnp.dot(m, wl_ref[...],
                                 preferred_element_type=jnp.float32) + bl_ref[...]

    return pl.pallas_call(
        body,
        grid=(nb,),
        in_specs=[pl.BlockSpec((BM, D), lambda i: (i, 0)),
                  pl.BlockSpec((BM, D), lambda i: (i, 0)),
                  pl.BlockSpec((BM, D), lambda i: (i, 0)),
                  pl.BlockSpec((BM, 1), lambda i: (i, 0)),
                  pl.BlockSpec((1, D), lambda i: (0, 0)),
                  pl.BlockSpec((D, D), lambda i: (0, 0)),
                  pl.BlockSpec((1, D), lambda i: (0, 0))],
        out_specs=pl.BlockSpec((1, D), lambda i: (0, 0)),
        out_shape=jax.ShapeDtypeStruct((1, D), jnp.float32),
        scratch_shapes=[pltpu.VMEM((1, D), jnp.float32)],
    )(q0, q1, g, dinv, b2, Wl, bl)


def kernel(x, edge_index, W1, b1, W2, b2, Wl, bl):
    src = edge_index[0]
    dst = edge_index[1]

    pad = EPAD - E
    pad_iota = lax.iota(jnp.int32, pad)
    src_p = jnp.concatenate([src, pad_iota % 128]).reshape(NW, NCH, CHUNK)
    dst_p = jnp.concatenate([dst, N + (pad_iota % (NPAD - N))]).reshape(NW, NCH, CHUNK)

    degp = _sc_degree(dst_p)
    h1 = _tc_matmul(x, W1)
    g1, dinv = _tc_scale(h1, degp[0, :N, None], degp[1, :N, None])
    p = _sc_spmm(g1, src_p, dst_p)
    g2 = _tc_layer(p[0, :N], p[1, :N], g1, dinv, b1.reshape(1, D), W2)
    q = _sc_spmm(g2, src_p, dst_p)
    out = _tc_final(q[0, :N], q[1, :N], g2, dinv, b2.reshape(1, D),
                    Wl, bl.reshape(1, D))
    return out[0]


# speedup vs baseline: 1.1994x; 1.0083x over previous
import dataclasses
import functools

import jax
import jax.numpy as jnp
from jax import lax
from jax.experimental import pallas as pl
from jax.experimental.pallas import tpu as pltpu
from jax.experimental.pallas import tpu_sc as plsc

N = 10000
E = 320000
D = 128

NC = 2
NS = 16
NW = NC * NS
CHUNK = 128
NCH = 80
EPW = NCH * CHUNK
EPAD = NW * EPW
NPAD = 10240
RPS = NPAD // NS

_mesh = plsc.VectorSubcoreMesh(core_axis_name="c", subcore_axis_name="s")

_cp_no_layout = pltpu.CompilerParams()
if "needs_layout_passes" in pltpu.CompilerParams.__dataclass_fields__:
    _cp_no_layout = dataclasses.replace(_cp_no_layout, needs_layout_passes=False)


def _shift_up(x, d, iota, sentinel):
    g = jax.lax.gather(
        x, jnp.maximum(iota - d, 0).reshape(16, 1),
        jax.lax.GatherDimensionNumbers(offset_dims=(), collapsed_slice_dims=(0,),
                                       start_index_map=(0,)),
        (1,), mode=jax.lax.GatherScatterMode.PROMISE_IN_BOUNDS)
    return jnp.where(iota >= d, g, sentinel)


@functools.partial(
    pl.kernel,
    out_type=jax.ShapeDtypeStruct((NC, NPAD), jnp.float32),
    mesh=_mesh,
    scratch_types=[
        pltpu.VMEM((NCH, CHUNK), jnp.int32),
        pltpu.VMEM((NPAD,), jnp.float32),
        pltpu.VMEM((NS, RPS), jnp.float32),
        pltpu.VMEM((RPS,), jnp.float32),
        pltpu.VMEM_SHARED((NS, NPAD), jnp.float32),
        pltpu.SemaphoreType.DMA,
    ],
    compiler_params=_cp_no_layout,
)
def _sc_degree(dst_hbm, out_hbm, dst_v, hist_v, blk_v, res_v, hist_sh, sem):
    cid = lax.axis_index("c")
    sid = lax.axis_index("s")
    wid = cid * NS + sid

    pltpu.async_copy(dst_hbm.at[wid], dst_v, sem)

    @pl.loop(0, NPAD, step=16)
    def _(i):
        hist_v[pl.ds(i, 16)] = jnp.zeros((16,), jnp.float32)

    pltpu.make_async_copy(dst_hbm.at[wid], dst_v, sem).wait()

    iota = lax.iota(jnp.int32, 16)
    ones = jnp.full((16,), 1.0, jnp.float32)

    @pl.loop(0, NCH)
    def _(j):
        @pl.loop(0, CHUNK, step=16)
        def _(k):
            idx = dst_v[j, pl.ds(k, 16)]
            sk, sv = plsc.sort_key_val(idx, ones)
            for dd in (1, 2, 4, 8):
                pk = _shift_up(sk, dd, iota, -1)
                pv = _shift_up(sv, dd, iota, 0.0)
                sv = sv + jnp.where(sk == pk, pv, 0.0)
            nk = jax.lax.gather(
                sk, jnp.minimum(iota + 1, 15).reshape(16, 1),
                jax.lax.GatherDimensionNumbers(
                    offset_dims=(), collapsed_slice_dims=(0,),
                    start_index_map=(0,)),
                (1,), mode=jax.lax.GatherScatterMode.PROMISE_IN_BOUNDS)
            last = jnp.where(iota == 15, jnp.int32(-2), nk) != sk
            plsc.addupdate_scatter(hist_v, [sk], sv, mask=last)

    pltpu.sync_copy(hist_v, hist_sh.at[sid])
    plsc.subcore_barrier()
    pltpu.sync_copy(hist_sh.at[:, pl.ds(sid * RPS, RPS)], blk_v)

    @pl.loop(0, RPS, step=16)
    def _(k):
        s = jnp.zeros((16,), jnp.float32)
        for h in range(NS):
            s = s + blk_v[h, pl.ds(k, 16)]
        res_v[pl.ds(k, 16)] = s

    pltpu.sync_copy(res_v, out_hbm.at[cid, pl.ds(sid * RPS, RPS)])


@functools.partial(
    pl.kernel,
    out_type=jax.ShapeDtypeStruct((NC, NPAD, D), jnp.float32),
    mesh=_mesh,
    scratch_types=[
        pltpu.VMEM((NCH // 2, CHUNK), jnp.int32),
        pltpu.VMEM((NCH // 2, CHUNK), jnp.int32),
        pltpu.VMEM((CHUNK, D), jnp.float32),
        pltpu.VMEM((CHUNK, D), jnp.float32),
        pltpu.VMEM_SHARED((NPAD, D), jnp.float32),
        pltpu.SemaphoreType.DMA,
        pltpu.SemaphoreType.DMA,
        pltpu.SemaphoreType.DMA,
    ],
)
def _sc_spmm(g_hbm, src_hbm, dst_hbm, out_hbm, src_v, dst_v, rows_a, rows_b,
             acc_sh, sem_a, sem_b, sem_i):
    cid = lax.axis_index("c")
    sid = lax.axis_index("s")
    wid = cid * NS + sid
    nchh = NCH // 2

    pltpu.async_copy(src_hbm.at[wid, pl.ds(0, nchh)], src_v, sem_i)
    pltpu.async_copy(dst_hbm.at[wid, pl.ds(0, nchh)], dst_v, sem_b)

    @pl.loop(0, CHUNK)
    def _(i):
        @pl.loop(0, D, step=16)
        def _(k):
            rows_a[i, pl.ds(k, 16)] = jnp.zeros((16,), jnp.float32)

    @pl.loop(0, RPS, step=CHUNK)
    def _(r):
        pltpu.async_copy(rows_a, acc_sh.at[pl.ds(sid * RPS + r, CHUNK)], sem_a)

    @pl.loop(0, RPS, step=CHUNK)
    def _(r):
        pltpu.make_async_copy(rows_a,
                              acc_sh.at[pl.ds(sid * RPS + r, CHUNK)],
                              sem_a).wait()

    pltpu.make_async_copy(src_hbm.at[wid, pl.ds(0, nchh)], src_v, sem_i).wait()
    pltpu.make_async_copy(dst_hbm.at[wid, pl.ds(0, nchh)], dst_v, sem_b).wait()
    plsc.subcore_barrier()

    for half in range(2):
        if half:
            pltpu.async_copy(src_hbm.at[wid, pl.ds(half * nchh, nchh)],
                             src_v, sem_i).wait()
            pltpu.async_copy(dst_hbm.at[wid, pl.ds(half * nchh, nchh)],
                             dst_v, sem_i).wait()
        pltpu.async_copy(g_hbm.at[src_v.at[0]], rows_a, sem_a)

        @pl.loop(0, nchh, step=2)
        def _(j):
            pltpu.make_async_copy(g_hbm.at[src_v.at[j]], rows_a, sem_a).wait()
            pltpu.async_copy(g_hbm.at[src_v.at[j + 1]], rows_b, sem_b)
            pltpu.sync_copy(rows_a, acc_sh.at[dst_v.at[j]], add=True)
            pltpu.make_async_copy(g_hbm.at[src_v.at[j + 1]], rows_b, sem_b).wait()

            @pl.when(j + 2 < nchh)
            def _():
                pltpu.async_copy(g_hbm.at[src_v.at[j + 2]], rows_a, sem_a)

            pltpu.sync_copy(rows_b, acc_sh.at[dst_v.at[j + 1]], add=True)

    plsc.subcore_barrier()
    pltpu.sync_copy(acc_sh.at[pl.ds(sid * RPS, RPS)],
                    out_hbm.at[cid, pl.ds(sid * RPS, RPS)])


BM = 1000


def _tc_matmul(x, W):
    def body(x_ref, w_ref, o_ref):
        o_ref[...] = jnp.dot(x_ref[...], w_ref[...],
                             preferred_element_type=jnp.float32)

    return pl.pallas_call(
        body,
        grid=(N // BM,),
        in_specs=[pl.BlockSpec((BM, D), lambda i: (i, 0)),
                  pl.BlockSpec((D, D), lambda i: (0, 0))],
        out_specs=pl.BlockSpec((BM, D), lambda i: (i, 0)),
        out_shape=jax.ShapeDtypeStruct((N, D), jnp.float32),
    )(x, W)


def _tc_scale(h1, d0, d1):
    def body(h_ref, d0_ref, d1_ref, g_ref, dinv_ref):
        dinv = lax.rsqrt(d0_ref[...] + d1_ref[...] + 1.0)
        dinv_ref[...] = dinv
        g_ref[...] = h_ref[...] * dinv

    return pl.pallas_call(
        body,
        grid=(N // BM,),
        in_specs=[pl.BlockSpec((BM, D), lambda i: (i, 0)),
                  pl.BlockSpec((BM, 1), lambda i: (i, 0)),
                  pl.BlockSpec((BM, 1), lambda i: (i, 0))],
        out_specs=[pl.BlockSpec((BM, D), lambda i: (i, 0)),
                   pl.BlockSpec((BM, 1), lambda i: (i, 0))],
        out_shape=[jax.ShapeDtypeStruct((N, D), jnp.float32),
                   jax.ShapeDtypeStruct((N, 1), jnp.float32)],
    )(h1, d0, d1)


def _tc_layer(p0, p1, g, dinv, b, W):
    def body(p0_ref, p1_ref, g_ref, dinv_ref, b_ref, w_ref, o_ref):
        y = (p0_ref[...] + p1_ref[...] + g_ref[...]) * dinv_ref[...] + b_ref[...]
        y = jnp.maximum(y, 0.0)
        o_ref[...] = jnp.dot(y, w_ref[...],
                             preferred_element_type=jnp.float32) * dinv_ref[...]

    return pl.pallas_call(
        body,
        grid=(N // BM,),
        in_specs=[pl.BlockSpec((BM, D), lambda i: (i, 0)),
                  pl.BlockSpec((BM, D), lambda i: (i, 0)),
                  pl.BlockSpec((BM, D), lambda i: (i, 0)),
                  pl.BlockSpec((BM, 1), lambda i: (i, 0)),
                  pl.BlockSpec((1, D), lambda i: (0, 0)),
                  pl.BlockSpec((D, D), lambda i: (0, 0))],
        out_specs=pl.BlockSpec((BM, D), lambda i: (i, 0)),
        out_shape=jax.ShapeDtypeStruct((N, D), jnp.float32),
    )(p0, p1, g, dinv, b, W)


def _tc_final(q0, q1, g, dinv, b2, Wl, bl):
    nb = N // BM

    def body(q0_ref, q1_ref, g_ref, dinv_ref, b2_ref, wl_ref, bl_ref, o_ref,
             acc_ref):
        i = pl.program_id(0)
        y = (q0_ref[...] + q1_ref[...] + g_ref[...]) * dinv_ref[...] + b2_ref[...]
        y = jnp.maximum(y, 0.0)
        s = jnp.sum(y, axis=0, keepdims=True)

        @pl.when(i == 0)
        def _():
            acc_ref[...] = s

        @pl.when(i > 0)
        def _():
            acc_ref[...] += s

        @pl.when(i == nb - 1)
        def _():
            m = acc_ref[...] * (1.0 / N)
            o_ref[...] = jnp.dot(m, wl_ref[...],
                                 preferred_element_type=jnp.float32) + bl_ref[...]

    return pl.pallas_call(
        body,
        grid=(nb,),
        in_specs=[pl.BlockSpec((BM, D), lambda i: (i, 0)),
                  pl.BlockSpec((BM, D), lambda i: (i, 0)),
                  pl.BlockSpec((BM, D), lambda i: (i, 0)),
                  pl.BlockSpec((BM, 1), lambda i: (i, 0)),
                  pl.BlockSpec((1, D), lambda i: (0, 0)),
                  pl.BlockSpec((D, D), lambda i: (0, 0)),
                  pl.BlockSpec((1, D), lambda i: (0, 0))],
        out_specs=pl.BlockSpec((1, D), lambda i: (0, 0)),
        out_shape=jax.ShapeDtypeStruct((1, D), jnp.float32),
        scratch_shapes=[pltpu.VMEM((1, D), jnp.float32)],
    )(q0, q1, g, dinv, b2, Wl, bl)


def kernel(x, edge_index, W1, b1, W2, b2, Wl, bl):
    src = edge_index[0]
    dst = edge_index[1]

    pad = EPAD - E
    pad_iota = lax.iota(jnp.int32, pad)
    src_p = jnp.concatenate([src, pad_iota % 128]).reshape(NW, NCH, CHUNK)
    dst_p = jnp.concatenate([dst, N + (pad_iota % (NPAD - N))]).reshape(NW, NCH, CHUNK)

    degp = _sc_degree(dst_p)
    h1 = _tc_matmul(x, W1)
    g1, dinv = _tc_scale(h1, degp[0, :N, None], degp[1, :N, None])
    p = _sc_spmm(g1, src_p, dst_p)
    g2 = _tc_layer(p[0, :N], p[1, :N], g1, dinv, b1.reshape(1, D), W2)
    q = _sc_spmm(g2, src_p, dst_p)
    out = _tc_final(q[0, :N], q[1, :N], g2, dinv, b2.reshape(1, D),
                    Wl, bl.reshape(1, D))
    return out[0]
